# baseline (speedup 1.0000x reference)
import functools

import jax
import jax.numpy as jnp
from jax import lax
from jax.experimental import pallas as pl
from jax.experimental.pallas import tpu as pltpu
from jax.experimental.pallas import tpu_sc as plsc

B = 16384
D = 32
CTX_SIZES = (100, 50, 24, 7)
CTX_DIM = 10
H1, H2 = 64, 32

NC = 2
NS = 16
NW = NC * NS
BPW = B // NW


def _sc_gather_body(uid_hbm, iid_hbm, U_hbm, I_hbm, ue_hbm, ie_hbm,
                    uidx_v, iidx_v, usem, isem):
    wid = lax.axis_index("s") * NC + lax.axis_index("c")
    base = wid * BPW
    pltpu.sync_copy(uid_hbm.at[pl.ds(base, BPW)], uidx_v)
    pltpu.sync_copy(iid_hbm.at[pl.ds(base, BPW)], iidx_v)
    lane = lax.iota(jnp.int32, 16)

    def issue(k, _):
        uvec = uidx_v[pl.ds(k * 16, 16)]
        ivec = iidx_v[pl.ds(k * 16, 16)]
        for l in range(16):
            uidx = uvec[l]
            iidx = ivec[l]
            pltpu.async_copy(U_hbm.at[pl.ds(uidx, 1)],
                             ue_hbm.at[pl.ds(base + k * 16 + l, 1)], usem)
            pltpu.async_copy(I_hbm.at[pl.ds(iidx, 1)],
                             ie_hbm.at[pl.ds(base + k * 16 + l, 1)], isem)
        return ()

    lax.fori_loop(0, BPW // 16, issue, ())

    def drain(j, _):
        pltpu.make_async_copy(U_hbm.at[pl.ds(0, 1)],
                              ue_hbm.at[pl.ds(base, 1)], usem).wait()
        pltpu.make_async_copy(I_hbm.at[pl.ds(0, 1)],
                              ie_hbm.at[pl.ds(base, 1)], isem).wait()
        return ()

    lax.fori_loop(0, BPW, drain, ())


_sc_gather = functools.partial(
    pl.kernel,
    mesh=plsc.VectorSubcoreMesh(core_axis_name="c", subcore_axis_name="s"),
    out_type=[
        jax.ShapeDtypeStruct((B, D), jnp.float32),
        jax.ShapeDtypeStruct((B, D), jnp.float32),
    ],
    scratch_types=[
        pltpu.VMEM((BPW,), jnp.int32),
        pltpu.VMEM((BPW,), jnp.int32),
        pltpu.SemaphoreType.DMA,
        pltpu.SemaphoreType.DMA,
    ],
)(_sc_gather_body)


RB = 2048


def _mlp_body(ue, ie, ctx, C0, C1, C2, C3, W1, b1, W2, b2, Wout, bout, out):
    ctx_i = ctx[...]
    feats = [ue[...], ie[...]]
    for j, (tbl, size) in enumerate(zip((C0, C1, C2, C3), CTX_SIZES)):
        ids = ctx_i[:, j:j + 1]
        onehot = (lax.broadcasted_iota(jnp.int32, (RB, size), 1) == ids)
        feats.append(jnp.dot(onehot.astype(jnp.float32), tbl[...],
                             preferred_element_type=jnp.float32))
    x = jnp.concatenate(feats, axis=1)
    h = jnp.maximum(jnp.dot(x, W1[...], preferred_element_type=jnp.float32)
                    + b1[...], 0.0)
    h = jnp.maximum(jnp.dot(h, W2[...], preferred_element_type=jnp.float32)
                    + b2[...], 0.0)
    z = jnp.dot(h, Wout[...], preferred_element_type=jnp.float32) + bout[...]
    out[...] = jax.nn.sigmoid(z)


def kernel(user_ids, item_ids, context_features, U, I, C0, C1, C2, C3,
           W1, b1, W2, b2, Wout, bout):
    user_ids = user_ids.astype(jnp.int32)
    item_ids = item_ids.astype(jnp.int32)
    ctx = context_features.astype(jnp.int32)

    ue, ie = _sc_gather(user_ids, item_ids, U, I)

    full = lambda shape: pl.BlockSpec(shape, lambda i: (0, 0))
    grid = B // RB
    out = pl.pallas_call(
        _mlp_body,
        grid=(grid,),
        in_specs=[
            pl.BlockSpec((RB, D), lambda i: (i, 0)),
            pl.BlockSpec((RB, D), lambda i: (i, 0)),
            pl.BlockSpec((RB, 4), lambda i: (i, 0)),
            full((CTX_SIZES[0], CTX_DIM)),
            full((CTX_SIZES[1], CTX_DIM)),
            full((CTX_SIZES[2], CTX_DIM)),
            full((CTX_SIZES[3], CTX_DIM)),
            full((2 * D + 4 * CTX_DIM, H1)),
            full((1, H1)),
            full((H1, H2)),
            full((1, H2)),
            full((H2, 1)),
            full((1, 1)),
        ],
        out_specs=pl.BlockSpec((RB, 1), lambda i: (i, 0)),
        out_shape=jax.ShapeDtypeStruct((B, 1), jnp.float32),
    )(ue, ie, ctx, C0, C1, C2, C3,
      W1, b1.reshape(1, H1), W2, b2.reshape(1, H2), Wout, bout.reshape(1, 1))
    return out


# speedup vs baseline: 1.7732x; 1.7732x over previous
import functools

import jax
import jax.numpy as jnp
from jax import lax
from jax.experimental import pallas as pl
from jax.experimental.pallas import tpu as pltpu
from jax.experimental.pallas import tpu_sc as plsc

B = 16384
D = 32
CTX_SIZES = (100, 50, 24, 7)
CTX_DIM = 10
H1, H2 = 64, 32

NC = 2
NS = 16
NW = NC * NS
BPW = B // NW
CH = 256


def _sc_gather_body(uid_hbm, iid_hbm, U_hbm, I_hbm, ue_hbm, ie_hbm,
                    uidx_v, iidx_v, urows_v, irows_v, usem, isem):
    wid = lax.axis_index("s") * NC + lax.axis_index("c")
    base = wid * BPW
    pltpu.sync_copy(uid_hbm.at[pl.ds(base, BPW)], uidx_v)
    pltpu.sync_copy(iid_hbm.at[pl.ds(base, BPW)], iidx_v)

    for chunk in range(BPW // CH):
        off = chunk * CH

        def issue(k, _):
            uvec = uidx_v[pl.ds(off + k * 16, 16)]
            ivec = iidx_v[pl.ds(off + k * 16, 16)]
            for l in range(16):
                j = k * 16 + l
                pltpu.async_copy(U_hbm.at[pl.ds(uvec[l], 1)],
                                 urows_v.at[pl.ds(j, 1)], usem)
                pltpu.async_copy(I_hbm.at[pl.ds(ivec[l], 1)],
                                 irows_v.at[pl.ds(j, 1)], isem)
            return ()

        lax.fori_loop(0, CH // 16, issue, ())

        def drain(j, _):
            pltpu.make_async_copy(U_hbm.at[pl.ds(0, 1)],
                                  urows_v.at[pl.ds(0, 1)], usem).wait()
            pltpu.make_async_copy(I_hbm.at[pl.ds(0, 1)],
                                  irows_v.at[pl.ds(0, 1)], isem).wait()
            return ()

        lax.fori_loop(0, CH, drain, ())
        pltpu.sync_copy(urows_v, ue_hbm.at[pl.ds(base + off, CH)])
        pltpu.sync_copy(irows_v, ie_hbm.at[pl.ds(base + off, CH)])


_sc_gather = functools.partial(
    pl.kernel,
    mesh=plsc.VectorSubcoreMesh(core_axis_name="c", subcore_axis_name="s"),
    out_type=[
        jax.ShapeDtypeStruct((B, D), jnp.float32),
        jax.ShapeDtypeStruct((B, D), jnp.float32),
    ],
    scratch_types=[
        pltpu.VMEM((BPW,), jnp.int32),
        pltpu.VMEM((BPW,), jnp.int32),
        pltpu.VMEM((CH, D), jnp.float32),
        pltpu.VMEM((CH, D), jnp.float32),
        pltpu.SemaphoreType.DMA,
        pltpu.SemaphoreType.DMA,
    ],
)(_sc_gather_body)


RB = 2048


def _mlp_body(ue, ie, ctx, C0, C1, C2, C3, W1, b1, W2, b2, Wout, bout, out):
    ctx_i = ctx[...]
    feats = [ue[...], ie[...]]
    for j, (tbl, size) in enumerate(zip((C0, C1, C2, C3), CTX_SIZES)):
        ids = ctx_i[:, j:j + 1]
        onehot = (lax.broadcasted_iota(jnp.int32, (RB, size), 1) == ids)
        feats.append(jnp.dot(onehot.astype(jnp.float32), tbl[...],
                             preferred_element_type=jnp.float32))
    x = jnp.concatenate(feats, axis=1)
    h = jnp.maximum(jnp.dot(x, W1[...], preferred_element_type=jnp.float32)
                    + b1[...], 0.0)
    h = jnp.maximum(jnp.dot(h, W2[...], preferred_element_type=jnp.float32)
                    + b2[...], 0.0)
    z = jnp.dot(h, Wout[...], preferred_element_type=jnp.float32) + bout[...]
    out[...] = jax.nn.sigmoid(z)


def kernel(user_ids, item_ids, context_features, U, I, C0, C1, C2, C3,
           W1, b1, W2, b2, Wout, bout):
    user_ids = user_ids.astype(jnp.int32)
    item_ids = item_ids.astype(jnp.int32)
    ctx = context_features.astype(jnp.int32)

    ue, ie = _sc_gather(user_ids, item_ids, U, I)

    full = lambda shape: pl.BlockSpec(shape, lambda i: (0, 0))
    grid = B // RB
    out = pl.pallas_call(
        _mlp_body,
        grid=(grid,),
        in_specs=[
            pl.BlockSpec((RB, D), lambda i: (i, 0)),
            pl.BlockSpec((RB, D), lambda i: (i, 0)),
            pl.BlockSpec((RB, 4), lambda i: (i, 0)),
            full((CTX_SIZES[0], CTX_DIM)),
            full((CTX_SIZES[1], CTX_DIM)),
            full((CTX_SIZES[2], CTX_DIM)),
            full((CTX_SIZES[3], CTX_DIM)),
            full((2 * D + 4 * CTX_DIM, H1)),
            full((1, H1)),
            full((H1, H2)),
            full((1, H2)),
            full((H2, 1)),
            full((1, 1)),
        ],
        out_specs=pl.BlockSpec((RB, 1), lambda i: (i, 0)),
        out_shape=jax.ShapeDtypeStruct((B, 1), jnp.float32),
    )(ue, ie, ctx, C0, C1, C2, C3,
      W1, b1.reshape(1, H1), W2, b2.reshape(1, H2), Wout, bout.reshape(1, 1))
    return out
